# baseline (speedup 1.0000x reference)
import functools

import jax
import jax.numpy as jnp
import numpy as np
from jax import lax
from jax.experimental import pallas as pl
from jax.experimental.pallas import tpu as pltpu
from jax.experimental.pallas import tpu_sc as plsc

VOCAB = 100000
D = 768
NC = 2
NS = 16
NW = NC * NS
CS = 16
NBUF = 6
GDEPTH = 3


def _pos_encoding(seq_len, d_model):
    pos = jnp.arange(seq_len, dtype=jnp.float32)[:, None]
    i = jnp.arange(0, d_model, 2, dtype=jnp.float32)
    div = jnp.exp(i * (-np.log(10000.0) / d_model))
    pe = jnp.zeros((seq_len, d_model), dtype=jnp.float32)
    pe = pe.at[:, 0::2].set(jnp.sin(pos * div))
    pe = pe.at[:, 1::2].set(jnp.cos(pos * div))
    return pe


def _make_kernel(batch, seq_len):
    bs = batch * seq_len
    s_per_w = seq_len // NW
    n_chunks = s_per_w // CS
    n_steps = n_chunks * batch
    mesh = plsc.VectorSubcoreMesh(core_axis_name="c", subcore_axis_name="s")

    @functools.partial(
        pl.kernel,
        mesh=mesh,
        out_type=jax.ShapeDtypeStruct((bs, D), jnp.float32),
        scratch_types=[
            pltpu.VMEM((batch, s_per_w), jnp.int32),
            [pltpu.VMEM((CS, D), jnp.float32) for _ in range(NBUF)],
            [pltpu.VMEM((CS, D), jnp.float32) for _ in range(2)],
            pltpu.VMEM_SHARED((NS, 2, CS, D), jnp.float32),
            [pltpu.SemaphoreType.DMA for _ in range(NBUF)],
            [pltpu.SemaphoreType.DMA for _ in range(2)],
            [pltpu.SemaphoreType.DMA for _ in range(2)],
        ],
    )
    def k(table_hbm, idx_hbm, pos_hbm, out_hbm,
          idx_all, rows, pos, stage, gsem, ssem, psem):
        sid = lax.axis_index("s")
        wid = sid * NC + lax.axis_index("c")
        s_base = wid * s_per_w

        for b in range(batch):
            pltpu.sync_copy(
                idx_hbm.at[pl.ds(b * seq_len + s_base, s_per_w)],
                idx_all.at[b])

        def start_gather(t):
            ci, b = divmod(t, batch)
            return pltpu.async_copy(
                table_hbm.at[idx_all.at[b, pl.ds(ci * CS, CS)]],
                rows[t % NBUF], gsem[t % NBUF])

        def start_pos(ci):
            return pltpu.async_copy(
                pos_hbm.at[pl.ds(s_base + ci * CS, CS)],
                pos[ci % 2], psem[ci % 2])

        pos_cp = start_pos(0)
        gathers = {t: start_gather(t) for t in range(GDEPTH)}
        stores = {}

        for t in range(n_steps):
            ci, b = divmod(t, batch)
            buf = t % NBUF
            if t + GDEPTH < n_steps:
                gathers[t + GDEPTH] = start_gather(t + GDEPTH)
            if b == 0:
                if ci + 1 < n_chunks:
                    nxt = start_pos(ci + 1)
                pos_cp.wait()
                if ci + 1 < n_chunks:
                    pos_cp = nxt
            gathers.pop(t).wait()

            rows_v, pos_v = rows[buf], pos[ci % 2]

            @pl.loop(0, CS)
            def row_body(r):
                @plsc.parallel_loop(0, D, step=16, unroll=8)
                def col_body(c):
                    rows_v[r, pl.ds(c, 16)] += pos_v[r, pl.ds(c, 16)]

            slot = t % 2
            if t - 2 in stores:
                stores.pop(t - 2).wait()
            pltpu.sync_copy(rows_v, stage.at[sid, slot])
            stores[t] = pltpu.async_copy(
                stage.at[sid, slot],
                out_hbm.at[pl.ds(b * seq_len + s_base + ci * CS, CS)],
                ssem[slot])

        for t in sorted(stores):
            stores.pop(t).wait()

    return k


@jax.jit
def kernel(x, table):
    batch, seq_len = x.shape
    pos = _pos_encoding(seq_len, D)
    idx = x.reshape(-1)
    out = _make_kernel(batch, seq_len)(table, idx, pos)
    return out.reshape(batch, seq_len, D)


# speedup vs baseline: 1.0841x; 1.0841x over previous
import functools

import jax
import jax.numpy as jnp
import numpy as np
from jax import lax
from jax.experimental import pallas as pl
from jax.experimental.pallas import tpu as pltpu
from jax.experimental.pallas import tpu_sc as plsc

VOCAB = 100000
D = 768
NC = 2
NS = 16
NW = NC * NS
CS = 16
NBUF = 8
GDEPTH = 4


def _pos_encoding(seq_len, d_model):
    pos = jnp.arange(seq_len, dtype=jnp.float32)[:, None]
    i = jnp.arange(0, d_model, 2, dtype=jnp.float32)
    div = jnp.exp(i * (-np.log(10000.0) / d_model))
    pe = jnp.zeros((seq_len, d_model), dtype=jnp.float32)
    pe = pe.at[:, 0::2].set(jnp.sin(pos * div))
    pe = pe.at[:, 1::2].set(jnp.cos(pos * div))
    return pe


def _make_kernel(batch, seq_len):
    bs = batch * seq_len
    s_per_w = seq_len // NW
    n_chunks = s_per_w // CS
    n_steps = n_chunks * batch
    mesh = plsc.VectorSubcoreMesh(core_axis_name="c", subcore_axis_name="s")

    @functools.partial(
        pl.kernel,
        mesh=mesh,
        out_type=jax.ShapeDtypeStruct((bs, D), jnp.float32),
        scratch_types=[
            pltpu.VMEM((batch, s_per_w), jnp.int32),
            [pltpu.VMEM((CS, D), jnp.float32) for _ in range(NBUF)],
            [pltpu.VMEM((CS, D), jnp.float32) for _ in range(2)],
            [pltpu.SemaphoreType.DMA for _ in range(NBUF)],
            [pltpu.SemaphoreType.DMA for _ in range(NBUF)],
            [pltpu.SemaphoreType.DMA for _ in range(2)],
        ],
    )
    def k(table_hbm, idx_hbm, pos_hbm, out_hbm,
          idx_all, rows, pos, gsem, ssem, psem):
        wid = lax.axis_index("s") * NC + lax.axis_index("c")
        s_base = wid * s_per_w

        for b in range(batch):
            pltpu.sync_copy(
                idx_hbm.at[pl.ds(b * seq_len + s_base, s_per_w)],
                idx_all.at[b])

        def start_gather(t):
            ci, b = divmod(t, batch)
            return pltpu.async_copy(
                table_hbm.at[idx_all.at[b, pl.ds(ci * CS, CS)]],
                rows[t % NBUF], gsem[t % NBUF])

        def start_pos(ci):
            return pltpu.async_copy(
                pos_hbm.at[pl.ds(s_base + ci * CS, CS)],
                pos[ci % 2], psem[ci % 2])

        pos_cp = start_pos(0)
        gathers = {t: start_gather(t) for t in range(GDEPTH)}
        stores = {}

        for t in range(n_steps):
            ci, b = divmod(t, batch)
            buf = t % NBUF
            if t + GDEPTH < n_steps:
                if t + GDEPTH - NBUF in stores:
                    stores.pop(t + GDEPTH - NBUF).wait()
                gathers[t + GDEPTH] = start_gather(t + GDEPTH)
            if b == 0:
                if ci + 1 < n_chunks:
                    nxt = start_pos(ci + 1)
                pos_cp.wait()
                if ci + 1 < n_chunks:
                    pos_cp = nxt
            gathers.pop(t).wait()

            rows_v, pos_v = rows[buf], pos[ci % 2]

            @pl.loop(0, CS)
            def row_body(r):
                @plsc.parallel_loop(0, D, step=16, unroll=8)
                def col_body(c):
                    rows_v[r, pl.ds(c, 16)] += pos_v[r, pl.ds(c, 16)]

            stores[t] = pltpu.async_copy(
                rows_v, out_hbm.at[pl.ds(b * seq_len + s_base + ci * CS, CS)],
                ssem[buf])

        for t in sorted(stores):
            stores.pop(t).wait()

    return k


@jax.jit
def kernel(x, table):
    batch, seq_len = x.shape
    pos = _pos_encoding(seq_len, D)
    idx = x.reshape(-1)
    out = _make_kernel(batch, seq_len)(table, idx, pos)
    return out.reshape(batch, seq_len, D)
